# baseline (speedup 1.0000x reference)
import functools
import math

import jax
import jax.numpy as jnp
from jax import lax
from jax.experimental import pallas as pl
from jax.experimental.pallas import tpu as pltpu
from jax.experimental.pallas import tpu_sc as plsc

D_MODEL = 64
PAD_W = 128
SCALE = math.sqrt(D_MODEL)
NUM_CORES = 2
NUM_SUBCORES = 16
NUM_WORKERS = NUM_CORES * NUM_SUBCORES
LANES = 16
BB = 128
NBUF = 4


def _pad_table(embedding_weight):
    vocab, d = embedding_weight.shape
    t = embedding_weight.T
    blk = 512

    def body(in_ref, out_ref):
        x = in_ref[...]
        out_ref[:, :d] = x.T * SCALE
        out_ref[:, d:] = jnp.zeros((blk, PAD_W - d), jnp.float32)

    return pl.pallas_call(
        body,
        grid=(pl.cdiv(vocab, blk),),
        in_specs=[pl.BlockSpec((d, blk), lambda i: (0, i))],
        out_specs=pl.BlockSpec((blk, PAD_W), lambda i: (i, 0)),
        out_shape=jax.ShapeDtypeStruct((vocab, PAD_W), jnp.float32),
    )(t)


def kernel(input_ids, embedding_weight):
    batch, seq = input_ids.shape
    vocab, d_model = embedding_weight.shape
    assert batch == NUM_WORKERS * BB and d_model == D_MODEL

    ids_flat = input_ids.reshape(batch * seq)
    table_fat = _pad_table(embedding_weight)

    mesh = plsc.VectorSubcoreMesh(
        core_axis_name="c", subcore_axis_name="s",
        num_cores=NUM_CORES, num_subcores=NUM_SUBCORES)

    @functools.partial(
        pl.kernel,
        mesh=mesh,
        out_type=jax.ShapeDtypeStruct((seq, D_MODEL, batch), jnp.float32),
        scratch_types=[
            pltpu.VMEM((BB * seq,), jnp.int32),
            pltpu.VMEM((NBUF, BB), jnp.int32),
            pltpu.VMEM((NBUF, BB, PAD_W), jnp.float32),
            pltpu.VMEM((NBUF, D_MODEL, BB), jnp.float32),
            [pltpu.SemaphoreType.DMA] * NBUF,
            [pltpu.SemaphoreType.DMA] * NBUF,
        ],
        compiler_params=pltpu.CompilerParams(
            use_tc_tiling_on_sc=True, needs_layout_passes=False),
    )
    def emb(ids_hbm, table_hbm, out_hbm, ids_v, idx_v, rows_v,
            plane_v, sem_g, sem_w):
        wid = lax.axis_index("s") * NUM_CORES + lax.axis_index("c")
        col0 = wid * BB

        pltpu.sync_copy(ids_hbm.at[pl.ds(col0 * seq, BB * seq)], ids_v)

        lane = lax.iota(jnp.int32, LANES)

        def build_idx(g, b):
            for bb0 in range(0, BB, LANES):
                pos = (lane + bb0) * seq + g
                idx_v[b, pl.ds(bb0, LANES)] = plsc.load_gather(ids_v, [pos])

        def issue_gather(g, b):
            build_idx(g, b)
            pltpu.async_copy(table_hbm.at[idx_v.at[b]], rows_v.at[b],
                             sem_g[b])

        def wait_gather(b):
            pltpu.make_async_copy(table_hbm.at[idx_v.at[b]], rows_v.at[b],
                                  sem_g[b]).wait()

        rot = [(lane + d) & (LANES - 1) for d in range(LANES)]
        cols = [c0 + lane for c0 in range(0, D_MODEL, LANES)]

        def transpose_scale(b):
            @plsc.parallel_loop(0, BB, LANES)
            def _(bb0):
                for d in range(LANES):
                    row_idx = bb0 + rot[d]
                    for ci, c0 in enumerate(range(0, D_MODEL, LANES)):
                        v = plsc.load_gather(rows_v.at[b],
                                             [row_idx, cols[ci]])
                        plsc.store_scatter(plane_v.at[b],
                                           [cols[ci], row_idx], v)

        def issue_write(g, b):
            pltpu.async_copy(plane_v.at[b],
                             out_hbm.at[g, :, pl.ds(col0, BB)], sem_w[b])

        def wait_write(g, b):
            pltpu.make_async_copy(plane_v.at[b],
                                  out_hbm.at[g, :, pl.ds(col0, BB)],
                                  sem_w[b]).wait()

        n = seq

        issue_gather(0, 0)
        issue_gather(1, 1)

        def outer(g0, carry):
            for db in range(NBUF):
                b = db
                gg = g0 * NBUF + db
                wait_gather(b)
                transpose_scale(b)
                issue_write(gg, b)
                nb = (b + 2) % NBUF

                @pl.when(gg + 2 < n)
                def _():
                    @pl.when(gg >= 2)
                    def _():
                        wait_write(gg - 2, nb)
                    issue_gather(gg + 2, nb)
            return carry

        lax.fori_loop(0, n // NBUF, outer, 0)

        for gg in range(n - 4, n):
            wait_write(gg, gg % NBUF)

    out_t = emb(ids_flat, table_fat)
    return out_t.transpose(2, 0, 1)


# speedup vs baseline: 1.6503x; 1.6503x over previous
import functools
import math

import jax
import jax.numpy as jnp
from jax import lax
from jax.experimental import pallas as pl
from jax.experimental.pallas import tpu as pltpu
from jax.experimental.pallas import tpu_sc as plsc

D_MODEL = 64
PAD_W = 128
SCALE = math.sqrt(D_MODEL)
NUM_CORES = 2
NUM_SUBCORES = 16
NUM_WORKERS = NUM_CORES * NUM_SUBCORES
LANES = 16
BB = 128
NBUF = 4


def kernel(input_ids, embedding_weight):
    batch, seq = input_ids.shape
    vocab, d_model = embedding_weight.shape
    assert batch == NUM_WORKERS * BB and d_model == D_MODEL

    ids_flat = input_ids.reshape(batch * seq)
    table_fat = jnp.pad(embedding_weight, ((0, 0), (0, PAD_W - D_MODEL)))

    mesh = plsc.VectorSubcoreMesh(
        core_axis_name="c", subcore_axis_name="s",
        num_cores=NUM_CORES, num_subcores=NUM_SUBCORES)

    @functools.partial(
        pl.kernel,
        mesh=mesh,
        out_type=jax.ShapeDtypeStruct((seq, D_MODEL, batch), jnp.float32),
        scratch_types=[
            pltpu.VMEM((BB * seq,), jnp.int32),
            pltpu.VMEM((NBUF, BB), jnp.int32),
            pltpu.VMEM((NBUF, BB, PAD_W), jnp.float32),
            pltpu.VMEM((NBUF, D_MODEL, BB), jnp.float32),
            [pltpu.SemaphoreType.DMA] * NBUF,
            [pltpu.SemaphoreType.DMA] * NBUF,
        ],
        compiler_params=pltpu.CompilerParams(
            use_tc_tiling_on_sc=True, needs_layout_passes=False),
    )
    def emb(ids_hbm, table_hbm, out_hbm, ids_v, idx_v, rows_v,
            plane_v, sem_g, sem_w):
        wid = lax.axis_index("s") * NUM_CORES + lax.axis_index("c")
        col0 = wid * BB

        pltpu.sync_copy(ids_hbm.at[pl.ds(col0 * seq, BB * seq)], ids_v)

        lane = lax.iota(jnp.int32, LANES)

        def build_idx(g, b):
            for bb0 in range(0, BB, LANES):
                pos = (lane + bb0) * seq + g
                idx_v[b, pl.ds(bb0, LANES)] = plsc.load_gather(ids_v, [pos])

        def issue_gather(g, b):
            build_idx(g, b)
            pltpu.async_copy(table_hbm.at[idx_v.at[b]], rows_v.at[b],
                             sem_g[b])

        def wait_gather(b):
            pltpu.make_async_copy(table_hbm.at[idx_v.at[b]], rows_v.at[b],
                                  sem_g[b]).wait()

        rot = [(lane + d) & (LANES - 1) for d in range(LANES)]
        cols = [c0 + lane for c0 in range(0, D_MODEL, LANES)]

        def transpose_scale(b):
            @plsc.parallel_loop(0, BB, LANES, unroll=2)
            def _(bb0):
                for d in range(LANES):
                    row_idx = bb0 + rot[d]
                    for ci, c0 in enumerate(range(0, D_MODEL, LANES)):
                        v = plsc.load_gather(rows_v.at[b],
                                             [row_idx, cols[ci]])
                        plsc.store_scatter(plane_v.at[b],
                                           [cols[ci], row_idx], v * SCALE)

        def issue_write(g, b):
            pltpu.async_copy(plane_v.at[b],
                             out_hbm.at[g, :, pl.ds(col0, BB)], sem_w[b])

        def wait_write(g, b):
            pltpu.make_async_copy(plane_v.at[b],
                                  out_hbm.at[g, :, pl.ds(col0, BB)],
                                  sem_w[b]).wait()

        n = seq

        issue_gather(0, 0)
        issue_gather(1, 1)

        def outer(g0, carry):
            for db in range(NBUF):
                b = db
                gg = g0 * NBUF + db
                wait_gather(b)
                transpose_scale(b)
                issue_write(gg, b)
                nb = (b + 2) % NBUF

                @pl.when(gg + 2 < n)
                def _():
                    @pl.when(gg >= 2)
                    def _():
                        wait_write(gg - 2, nb)
                    issue_gather(gg + 2, nb)
            return carry

        lax.fori_loop(0, n // NBUF, outer, 0)

        for gg in range(n - 4, n):
            wait_write(gg, gg % NBUF)

    out_t = emb(ids_flat, table_fat)
    return out_t.transpose(2, 0, 1)


# speedup vs baseline: 1.8920x; 1.1465x over previous
import functools
import math

import jax
import jax.numpy as jnp
from jax import lax
from jax.experimental import pallas as pl
from jax.experimental.pallas import tpu as pltpu
from jax.experimental.pallas import tpu_sc as plsc

D_MODEL = 64
PAD_W = 128
SCALE = math.sqrt(D_MODEL)
NUM_CORES = 2
NUM_SUBCORES = 16
NUM_WORKERS = NUM_CORES * NUM_SUBCORES
LANES = 16
BB = 128
NBUF = 4


def kernel(input_ids, embedding_weight):
    batch, seq = input_ids.shape
    vocab, d_model = embedding_weight.shape
    assert batch == NUM_WORKERS * BB and d_model == D_MODEL

    ids_flat = input_ids.reshape(batch * seq)
    table_fat = jnp.pad(embedding_weight, ((0, 0), (0, PAD_W - D_MODEL)))

    mesh = plsc.VectorSubcoreMesh(
        core_axis_name="c", subcore_axis_name="s",
        num_cores=NUM_CORES, num_subcores=NUM_SUBCORES)

    @functools.partial(
        pl.kernel,
        mesh=mesh,
        out_type=jax.ShapeDtypeStruct((seq, D_MODEL, batch), jnp.float32),
        scratch_types=[
            pltpu.VMEM((BB * seq,), jnp.int32),
            pltpu.VMEM((NBUF, BB), jnp.int32),
            pltpu.VMEM((NBUF, BB, PAD_W), jnp.float32),
            pltpu.VMEM((NBUF, D_MODEL, BB), jnp.float32),
            [pltpu.SemaphoreType.DMA] * NBUF,
            [pltpu.SemaphoreType.DMA] * NBUF,
        ],
        compiler_params=pltpu.CompilerParams(
            use_tc_tiling_on_sc=True, needs_layout_passes=False),
    )
    def emb(ids_hbm, table_hbm, out_hbm, ids_v, idx_v, rows_v,
            plane_v, sem_g, sem_w):
        wid = lax.axis_index("s") * NUM_CORES + lax.axis_index("c")
        col0 = wid * BB

        pltpu.sync_copy(ids_hbm.at[pl.ds(col0 * seq, BB * seq)], ids_v)

        lane = lax.iota(jnp.int32, LANES)

        def build_idx(g, b):
            for bb0 in range(0, BB, LANES):
                pos = (lane + bb0) * seq + g
                idx_v[b, pl.ds(bb0, LANES)] = plsc.load_gather(ids_v, [pos])

        def issue_gather(g, b):
            build_idx(g, b)
            pltpu.async_copy(table_hbm.at[idx_v.at[b]], rows_v.at[b],
                             sem_g[b])

        def wait_gather(b):
            pltpu.make_async_copy(table_hbm.at[idx_v.at[b]], rows_v.at[b],
                                  sem_g[b]).wait()

        rot = [(lane + d) & (LANES - 1) for d in range(LANES)]
        cols = [c0 + lane for c0 in range(0, D_MODEL, LANES)]

        def transpose_scale(b):
            @plsc.parallel_loop(0, BB, LANES, unroll=4)
            def _(bb0):
                for d in range(LANES):
                    row_idx = bb0 + rot[d]
                    for ci, c0 in enumerate(range(0, D_MODEL, LANES)):
                        v = plsc.load_gather(rows_v.at[b],
                                             [row_idx, cols[ci]])
                        plsc.store_scatter(plane_v.at[b],
                                           [cols[ci], row_idx], v * SCALE)

        def issue_write(g, b):
            pltpu.async_copy(plane_v.at[b],
                             out_hbm.at[g, :, pl.ds(col0, BB)], sem_w[b])

        def wait_write(g, b):
            pltpu.make_async_copy(plane_v.at[b],
                                  out_hbm.at[g, :, pl.ds(col0, BB)],
                                  sem_w[b]).wait()

        n = seq

        issue_gather(0, 0)
        issue_gather(1, 1)

        def outer(g0, carry):
            for db in range(NBUF):
                b = db
                gg = g0 * NBUF + db
                wait_gather(b)
                transpose_scale(b)
                issue_write(gg, b)
                nb = (b + 2) % NBUF

                @pl.when(gg + 2 < n)
                def _():
                    @pl.when(gg >= 2)
                    def _():
                        wait_write(gg - 2, nb)
                    issue_gather(gg + 2, nb)
            return carry

        lax.fori_loop(0, n // NBUF, outer, 0)

        for gg in range(n - 4, n):
            wait_write(gg, gg % NBUF)

    out_t = emb(ids_flat, table_fat)
    return out_t.transpose(2, 0, 1)


# speedup vs baseline: 1.8995x; 1.0039x over previous
import functools
import math

import jax
import jax.numpy as jnp
from jax import lax
from jax.experimental import pallas as pl
from jax.experimental.pallas import tpu as pltpu
from jax.experimental.pallas import tpu_sc as plsc

D_MODEL = 64
PAD_W = 128
SCALE = math.sqrt(D_MODEL)
NUM_CORES = 2
NUM_SUBCORES = 16
NUM_WORKERS = NUM_CORES * NUM_SUBCORES
LANES = 16
BB = 128
NBUF = 4


def kernel(input_ids, embedding_weight):
    batch, seq = input_ids.shape
    vocab, d_model = embedding_weight.shape
    assert batch == NUM_WORKERS * BB and d_model == D_MODEL

    ids_flat = input_ids.reshape(batch * seq)
    table_fat = jnp.pad(embedding_weight, ((0, 0), (0, PAD_W - D_MODEL)))

    mesh = plsc.VectorSubcoreMesh(
        core_axis_name="c", subcore_axis_name="s",
        num_cores=NUM_CORES, num_subcores=NUM_SUBCORES)

    @functools.partial(
        pl.kernel,
        mesh=mesh,
        out_type=jax.ShapeDtypeStruct((seq, D_MODEL, batch), jnp.float32),
        scratch_types=[
            pltpu.VMEM((BB * seq,), jnp.int32),
            pltpu.VMEM((NBUF, BB), jnp.int32),
            pltpu.VMEM((NBUF, BB, PAD_W), jnp.float32),
            pltpu.VMEM((2, D_MODEL, BB), jnp.float32),
            [pltpu.SemaphoreType.DMA] * NBUF,
            [pltpu.SemaphoreType.DMA] * 2,
        ],
        compiler_params=pltpu.CompilerParams(
            use_tc_tiling_on_sc=True, needs_layout_passes=False),
    )
    def emb(ids_hbm, table_hbm, out_hbm, ids_v, idx_v, rows_v,
            plane_v, sem_g, sem_w):
        wid = lax.axis_index("s") * NUM_CORES + lax.axis_index("c")
        col0 = wid * BB

        pltpu.sync_copy(ids_hbm.at[pl.ds(col0 * seq, BB * seq)], ids_v)

        lane = lax.iota(jnp.int32, LANES)

        def build_idx(g, b):
            for bb0 in range(0, BB, LANES):
                pos = (lane + bb0) * seq + g
                idx_v[b, pl.ds(bb0, LANES)] = plsc.load_gather(ids_v, [pos])

        def issue_gather(g, b):
            build_idx(g, b)
            pltpu.async_copy(table_hbm.at[idx_v.at[b]], rows_v.at[b],
                             sem_g[b])

        def wait_gather(b):
            pltpu.make_async_copy(table_hbm.at[idx_v.at[b]], rows_v.at[b],
                                  sem_g[b]).wait()

        rot = [(lane + d) & (LANES - 1) for d in range(LANES)]
        cols = [c0 + lane for c0 in range(0, D_MODEL, LANES)]

        def transpose_scale(b, pb):
            @plsc.parallel_loop(0, BB, LANES, unroll=4)
            def _(bb0):
                for d in range(LANES):
                    row_idx = bb0 + rot[d]
                    for ci, c0 in enumerate(range(0, D_MODEL, LANES)):
                        v = plsc.load_gather(rows_v.at[b],
                                             [row_idx, cols[ci]])
                        plsc.store_scatter(plane_v.at[pb],
                                           [cols[ci], row_idx], v * SCALE)

        def issue_write(g, pb):
            pltpu.async_copy(plane_v.at[pb],
                             out_hbm.at[g, :, pl.ds(col0, BB)], sem_w[pb])

        def wait_write(g, pb):
            pltpu.make_async_copy(plane_v.at[pb],
                                  out_hbm.at[g, :, pl.ds(col0, BB)],
                                  sem_w[pb]).wait()

        n = seq

        issue_gather(0, 0)
        issue_gather(1, 1)

        def outer(g0, carry):
            for db in range(NBUF):
                b = db
                pb = db % 2
                gg = g0 * NBUF + db
                wait_gather(b)

                @pl.when(gg >= 2)
                def _():
                    wait_write(gg - 2, pb)

                transpose_scale(b, pb)
                issue_write(gg, pb)

                @pl.when(gg + 2 < n)
                def _():
                    issue_gather(gg + 2, (b + 2) % NBUF)
            return carry

        lax.fori_loop(0, n // NBUF, outer, 0)

        for gg in (n - 2, n - 1):
            wait_write(gg, gg % 2)

    out_t = emb(ids_flat, table_fat)
    return out_t.transpose(2, 0, 1)


# speedup vs baseline: 1.9899x; 1.0476x over previous
import functools
import math

import jax
import jax.numpy as jnp
from jax import lax
from jax.experimental import pallas as pl
from jax.experimental.pallas import tpu as pltpu
from jax.experimental.pallas import tpu_sc as plsc

D_MODEL = 64
PAD_W = 128
SCALE = math.sqrt(D_MODEL)
NUM_CORES = 2
NUM_SUBCORES = 16
NUM_WORKERS = NUM_CORES * NUM_SUBCORES
LANES = 16
BB = 128
NBUF = 4


def kernel(input_ids, embedding_weight):
    batch, seq = input_ids.shape
    vocab, d_model = embedding_weight.shape
    assert batch == NUM_WORKERS * BB and d_model == D_MODEL

    ids_flat = input_ids.reshape(batch * seq)
    table_fat = jnp.pad(embedding_weight, ((0, 0), (0, PAD_W - D_MODEL)))

    mesh = plsc.VectorSubcoreMesh(
        core_axis_name="c", subcore_axis_name="s",
        num_cores=NUM_CORES, num_subcores=NUM_SUBCORES)

    @functools.partial(
        pl.kernel,
        mesh=mesh,
        out_type=jax.ShapeDtypeStruct((seq, D_MODEL, batch), jnp.float32),
        scratch_types=[
            pltpu.VMEM((BB * seq,), jnp.int32),
            pltpu.VMEM((NBUF, BB), jnp.int32),
            pltpu.VMEM((NBUF, BB, PAD_W), jnp.float32),
            pltpu.VMEM((2, D_MODEL, BB), jnp.float32),
            [pltpu.SemaphoreType.DMA] * NBUF,
            [pltpu.SemaphoreType.DMA] * 2,
        ],
        compiler_params=pltpu.CompilerParams(
            use_tc_tiling_on_sc=True, needs_layout_passes=False),
    )
    def emb(ids_hbm, table_hbm, out_hbm, ids_v, idx_v, rows_v,
            plane_v, sem_g, sem_w):
        wid = lax.axis_index("s") * NUM_CORES + lax.axis_index("c")
        col0 = wid * BB

        pltpu.sync_copy(ids_hbm.at[pl.ds(col0 * seq, BB * seq)], ids_v)

        lane = lax.iota(jnp.int32, LANES)

        def build_idx(g, b):
            for bb0 in range(0, BB, LANES):
                pos = (lane + bb0) * seq + g
                idx_v[b, pl.ds(bb0, LANES)] = plsc.load_gather(ids_v, [pos])

        def issue_gather(g, b):
            build_idx(g, b)
            pltpu.async_copy(table_hbm.at[idx_v.at[b]], rows_v.at[b],
                             sem_g[b])

        def wait_gather(b):
            pltpu.make_async_copy(table_hbm.at[idx_v.at[b]], rows_v.at[b],
                                  sem_g[b]).wait()

        rot = [(lane + d) & (LANES - 1) for d in range(LANES)]
        cols = [c0 + lane for c0 in range(0, D_MODEL, LANES)]

        def transpose_scale(b, pb):
            @plsc.parallel_loop(0, BB, LANES, unroll=4)
            def _(bb0):
                for d in range(LANES):
                    row_idx = bb0 + rot[d]
                    for ci, c0 in enumerate(range(0, D_MODEL, LANES)):
                        v = plsc.load_gather(rows_v.at[b],
                                             [row_idx, cols[ci]])
                        plsc.store_scatter(plane_v.at[pb],
                                           [cols[ci], row_idx], v * SCALE)

        def issue_write(g, pb):
            pltpu.async_copy(plane_v.at[pb],
                             out_hbm.at[g, :, pl.ds(col0, BB)], sem_w[pb])

        def wait_write(g, pb):
            pltpu.make_async_copy(plane_v.at[pb],
                                  out_hbm.at[g, :, pl.ds(col0, BB)],
                                  sem_w[pb]).wait()

        n = seq

        issue_gather(0, 0)
        issue_gather(1, 1)
        issue_gather(2, 2)

        def outer(g0, carry):
            for db in range(NBUF):
                b = db
                pb = db % 2
                gg = g0 * NBUF + db
                wait_gather(b)

                @pl.when(gg >= 2)
                def _():
                    wait_write(gg - 2, pb)

                transpose_scale(b, pb)
                issue_write(gg, pb)

                @pl.when(gg + 3 < n)
                def _():
                    issue_gather(gg + 3, (b + 3) % NBUF)
            return carry

        lax.fori_loop(0, n // NBUF, outer, 0)

        for gg in (n - 2, n - 1):
            wait_write(gg, gg % 2)

    out_t = emb(ids_flat, table_fat)
    return out_t.transpose(2, 0, 1)
